# baseline (speedup 1.0000x reference)
import functools

import jax
import jax.numpy as jnp
from jax import lax
from jax.experimental import pallas as pl
from jax.experimental.pallas import tpu as pltpu
from jax.experimental.pallas import tpu_sc as plsc

_B = 8192
_K = 8192
_D = 64
_BM = 256


_CHUNK = 2048


def _rne_bf16(v):
    u = lax.bitcast_convert_type(v, jnp.int32)
    u = (u + jnp.int32(0x7FFF) + ((u >> 16) & jnp.int32(1))) \
        & jnp.int32(-65536)
    return lax.bitcast_convert_type(u, jnp.float32)


def _dist_body(z_ref, cb_ref, z2_ref, c2_ref, idx_ref, loss_ref):
    i = pl.program_id(0)
    z = z_ref[...]
    cb = cb_ref[...]
    z2 = z2_ref[...]
    c2 = c2_ref[...]
    zc = lax.dot_general(z, cb, (((1,), (1,)), ((), ())),
                         preferred_element_type=jnp.float32)
    d2 = z2 + c2 - 2.0 * zc
    dists = jnp.sqrt(jnp.clip(d2, 0.0, None))

    acc_v = acc_e = acc_i = None
    for c in range(_K // _CHUNK):
        seg = dists[:, c * _CHUNK:(c + 1) * _CHUNK]
        m = jnp.min(seg, axis=1, keepdims=True)
        iota = lax.broadcasted_iota(jnp.int32, seg.shape, 1)
        ci = jnp.min(jnp.where(seg == m, iota, _K), axis=1,
                     keepdims=True) + c * _CHUNK
        if acc_v is None:
            acc_v, acc_e, acc_i = _rne_bf16(m), m, ci
        else:
            take = m < acc_v
            acc_i = jnp.where(take, ci, acc_i)
            acc_e = jnp.where(take, m, acc_e)
            acc_v = _rne_bf16(jnp.where(take, m, acc_v))
    idx_ref[...] = acc_i

    @pl.when(i == 0)
    def _():
        loss_ref[...] = jnp.zeros((1, 1), jnp.float32)

    loss_ref[...] += jnp.sum(acc_e * acc_e, keepdims=True)


_dist_call = pl.pallas_call(
    _dist_body,
    grid=(_B // _BM,),
    in_specs=[
        pl.BlockSpec((_BM, _D), lambda i: (i, 0)),
        pl.BlockSpec((_K, _D), lambda i: (0, 0)),
        pl.BlockSpec((_BM, 1), lambda i: (i, 0)),
        pl.BlockSpec((1, _K), lambda i: (0, 0)),
    ],
    out_specs=[
        pl.BlockSpec((_BM, 1), lambda i: (i, 0)),
        pl.BlockSpec((1, 1), lambda i: (0, 0)),
    ],
    out_shape=[
        jax.ShapeDtypeStruct((_B, 1), jnp.int32),
        jax.ShapeDtypeStruct((1, 1), jnp.float32),
    ],
)


_DP = 128


@functools.cache
def _make_gather():
    info = plsc.get_sparse_core_info()
    nc, ns = info.num_cores, info.num_subcores
    nw = nc * ns
    b_per_w = _B // nw
    mesh = plsc.VectorSubcoreMesh(core_axis_name="c", subcore_axis_name="s")

    @functools.partial(
        pl.kernel, mesh=mesh,
        out_type=jax.ShapeDtypeStruct((_B, _DP), jnp.float32),
        scratch_types=[
            pltpu.VMEM((b_per_w,), jnp.int32),
            pltpu.VMEM((b_per_w, _DP), jnp.float32),
            pltpu.SemaphoreType.DMA,
        ],
    )
    def gather_k(cb_hbm, idx_hbm, out_hbm, idx_v, rows_v, sem):
        wid = lax.axis_index("s") * nc + lax.axis_index("c")
        base = wid * b_per_w
        pltpu.sync_copy(idx_hbm.at[pl.ds(base, b_per_w)], idx_v)
        pltpu.async_copy(cb_hbm.at[idx_v], rows_v, sem).wait()
        pltpu.sync_copy(rows_v, out_hbm.at[pl.ds(base, b_per_w)])

    return gather_k


def kernel(z, codebook):
    z2 = jnp.sum(z * z, axis=1, keepdims=True)
    c2 = jnp.sum(codebook * codebook, axis=1)[None, :]
    idx2d, loss_sum = _dist_call(z, codebook, z2, c2)
    indices = idx2d.reshape(_B)
    cb_pad = jnp.concatenate([codebook, jnp.zeros_like(codebook)], axis=1)
    z_q = _make_gather()(cb_pad, indices)[:, :_D]
    commit_loss = (loss_sum[0, 0] / (_B * _D)).astype(jnp.float32)
    return (z_q, indices, commit_loss)


# speedup vs baseline: 1.2351x; 1.2351x over previous
import functools

import jax
import jax.numpy as jnp
from jax import lax
from jax.experimental import pallas as pl
from jax.experimental.pallas import tpu as pltpu
from jax.experimental.pallas import tpu_sc as plsc

_B = 8192
_K = 8192
_D = 64
_BM = 256
_CHUNK = 2048
_WIDE = 1.0 + 2.0 ** -17


def _rne_bf16(v):
    u = lax.bitcast_convert_type(v, jnp.int32)
    u = (u + jnp.int32(0x7FFF) + ((u >> 16) & jnp.int32(1))) \
        & jnp.int32(-65536)
    return lax.bitcast_convert_type(u, jnp.float32)


def _dist_body(zz_ref, cb_ref, z2_ref, c2_ref, idx_ref, loss_ref,
               sv_ref, si_ref):
    i = pl.program_id(0)
    zz = zz_ref[...]
    cb = cb_ref[...]
    z2 = z2_ref[...]
    c2 = c2_ref[...]
    zc2 = lax.dot_general(zz, cb, (((1,), (1,)), ((), ())),
                          preferred_element_type=jnp.float32)
    sub = (z2 + c2) - zc2
    iota = lax.broadcasted_iota(jnp.int32, (_BM, _CHUNK), 1)

    acc_v = acc_e = acc_i = None
    for c in range(_K // _CHUNK):
        seg = sub[:, c * _CHUNK:(c + 1) * _CHUNK]
        md2 = jnp.min(seg, axis=1, keepdims=True)
        mx = jnp.maximum(md2, 0.0)
        thr = mx * jnp.float32(_WIDE) + jnp.float32(1e-37)
        q = jnp.where(seg <= thr, iota, _CHUNK)
        minq = jnp.min(q, axis=1, keepdims=True)
        sumq = jnp.sum(q, axis=1, keepdims=True)
        singleton = (sumq - minq) == (_CHUNK - 1) * _CHUNK
        sv_ref[...] = jnp.sqrt(mx)
        si_ref[...] = minq + c * _CHUNK

        @pl.when(jnp.logical_not(jnp.all(singleton)))
        def _(seg=seg):
            dists = jnp.maximum(seg, 0.0)
            dists = jnp.sqrt(dists)
            m = jnp.min(dists, axis=1, keepdims=True)
            ci = jnp.min(jnp.where(dists == m, iota, _CHUNK), axis=1,
                         keepdims=True)
            sv_ref[...] = m
            si_ref[...] = ci + c * _CHUNK

        m_c = sv_ref[...]
        i_c = si_ref[...]
        if acc_v is None:
            acc_v, acc_e, acc_i = _rne_bf16(m_c), m_c, i_c
        else:
            take = m_c < acc_v
            acc_i = jnp.where(take, i_c, acc_i)
            acc_e = jnp.where(take, m_c, acc_e)
            acc_v = _rne_bf16(jnp.where(take, m_c, acc_v))
    idx_ref[...] = acc_i

    @pl.when(i == 0)
    def _():
        loss_ref[...] = jnp.zeros((1, 1), jnp.float32)

    loss_ref[...] += jnp.sum(acc_e * acc_e, keepdims=True)


_dist_call = pl.pallas_call(
    _dist_body,
    grid=(_B // _BM,),
    in_specs=[
        pl.BlockSpec((_BM, _D), lambda i: (i, 0)),
        pl.BlockSpec((_K, _D), lambda i: (0, 0)),
        pl.BlockSpec((_BM, 1), lambda i: (i, 0)),
        pl.BlockSpec((1, _K), lambda i: (0, 0)),
    ],
    out_specs=[
        pl.BlockSpec((_BM, 1), lambda i: (i, 0)),
        pl.BlockSpec((1, 1), lambda i: (0, 0)),
    ],
    out_shape=[
        jax.ShapeDtypeStruct((_B, 1), jnp.int32),
        jax.ShapeDtypeStruct((1, 1), jnp.float32),
    ],
    scratch_shapes=[
        pltpu.VMEM((_BM, 1), jnp.float32),
        pltpu.VMEM((_BM, 1), jnp.int32),
    ],
)

_DP = 128


@functools.cache
def _make_gather():
    info = plsc.get_sparse_core_info()
    nc, ns = info.num_cores, info.num_subcores
    nw = nc * ns
    b_per_w = _B // nw
    mesh = plsc.VectorSubcoreMesh(core_axis_name="c", subcore_axis_name="s")

    @functools.partial(
        pl.kernel, mesh=mesh,
        out_type=jax.ShapeDtypeStruct((_B, _DP), jnp.float32),
        scratch_types=[
            pltpu.VMEM((b_per_w,), jnp.int32),
            pltpu.VMEM((b_per_w, _DP), jnp.float32),
            pltpu.SemaphoreType.DMA,
        ],
    )
    def gather_k(cb_hbm, idx_hbm, out_hbm, idx_v, rows_v, sem):
        wid = lax.axis_index("s") * nc + lax.axis_index("c")
        base = wid * b_per_w
        pltpu.sync_copy(idx_hbm.at[pl.ds(base, b_per_w)], idx_v)
        pltpu.async_copy(cb_hbm.at[idx_v], rows_v, sem).wait()
        pltpu.sync_copy(rows_v, out_hbm.at[pl.ds(base, b_per_w)])

    return gather_k


def kernel(z, codebook):
    z2 = jnp.sum(z * z, axis=1, keepdims=True)
    c2 = jnp.sum(codebook * codebook, axis=1)[None, :]
    idx2d, loss_sum = _dist_call(2.0 * z, codebook, z2, c2)
    indices = idx2d.reshape(_B)
    cb_pad = jnp.concatenate([codebook, jnp.zeros_like(codebook)], axis=1)
    z_q = _make_gather()(cb_pad, indices)[:, :_D]
    commit_loss = (loss_sum[0, 0] / (_B * _D)).astype(jnp.float32)
    return (z_q, indices, commit_loss)
